# baseline (speedup 1.0000x reference)
import functools

import jax
import jax.numpy as jnp
from jax import lax
from jax.experimental import pallas as pl
from jax.experimental.pallas import tpu as pltpu
from jax.experimental.pallas import tpu_sc as plsc

U = 100000
E = 32
NE = 1_000_000
B = 16384
MOM = 0.05
NS = 16
C = 1024

EDGE_FULL = NE // C
EDGE_PER_TILE = EDGE_FULL // NS
EDGE_REM = NE - EDGE_FULL * C
NODE_FULL = U // C
NODE_K = -(-NODE_FULL // NS)
NODE_REM = U - NODE_FULL * C

NRANGE = 4
RANGE = U // NRANGE
DUMMY = RANGE
ACC_ROWS = RANGE + 24
CAP_CHUNKS = 64
REGION = CAP_CHUNKS * C
WB = 512
WB_FULL = RANGE // WB
WB_K = WB_FULL // NS
WB_REM = RANGE - WB_FULL * WB

CNT_STRIDE = 8
CNT_LEN = NRANGE * NS * CNT_STRIDE + 8

_f32 = jnp.float32
_i32 = jnp.int32


@functools.lru_cache(maxsize=None)
def _mesh():
    return plsc.VectorSubcoreMesh(core_axis_name="c", subcore_axis_name="s",
                                  num_cores=2, num_subcores=NS)


_params = pltpu.CompilerParams(needs_layout_passes=False,
                               use_tc_tiling_on_sc=False)


def _rsqrt16(d):
    xi = plsc.bitcast(d, _i32)
    y = plsc.bitcast(jnp.int32(0x5F3759DF) - (xi >> 1), _f32)
    for _ in range(3):
        y = y * (1.5 - 0.5 * d * y * y)
    return y


def _lane_bcast(v, j):
    idx = jnp.full((16, 1), j, dtype=_i32)
    dnums = lax.GatherDimensionNumbers(
        offset_dims=(), collapsed_slice_dims=(0,), start_index_map=(0,))
    return lax.gather(v, idx, dnums, (1,),
                      mode=lax.GatherScatterMode.PROMISE_IN_BOUNDS)


def _fill(ref, n, value):
    val = jnp.full((16,), value, _f32)

    def body(v, carry):
        ref[pl.ds(v * 16, 16)] = val
        return carry

    lax.fori_loop(0, n // 16, body, 0)


def _zero_rows(ref, rows):
    z = jnp.zeros((16,), _f32)

    def body(r, carry):
        ref[r, pl.ds(0, 16)] = z
        ref[r, pl.ds(16, 16)] = z
        return carry

    lax.fori_loop(0, rows, body, 0)


def _k1_body(users, items, user_emb, item_emb, edge_u, edge_i,
             a_u, a_i, h0_u, h0_i, m_u, m_i,
             csrc_u, crel_u, cnt_u, csrc_i, crel_i, cnt_i,
             deg_sp, m_sp, dstb, srcb, fb, dstr, srcr, oner,
             degb, ab, rowb, mb, stg_s, stg_r, cntb):
    c = lax.axis_index("c")
    s = lax.axis_index("s")

    def side(bidx, edge_dst, edge_src, emb, a_out, h_out, m_out,
             csrc, crel, cnt):
        _fill(fb, C, 0.0)

        def zb(k, carry):
            g = k * NS + s

            @pl.when(g < NODE_FULL)
            def _():
                pltpu.sync_copy(fb, deg_sp.at[pl.ds(g * C, C)])
                pltpu.sync_copy(fb, m_sp.at[pl.ds(g * C, C)])

            return carry

        lax.fori_loop(0, NODE_K, zb, 0)

        @pl.when(s == NS - 1)
        def _():
            pltpu.sync_copy(fb.at[pl.ds(0, NODE_REM)],
                            deg_sp.at[pl.ds(NODE_FULL * C, NODE_REM)])
            pltpu.sync_copy(fb.at[pl.ds(0, NODE_REM)],
                            m_sp.at[pl.ds(NODE_FULL * C, NODE_REM)])

        plsc.subcore_barrier()

        _fill(fb, C, 1.0)
        _fill(oner, EDGE_REM, 1.0)

        pltpu.sync_copy(bidx.at[pl.ds(s * C, C)], dstb)
        pltpu.sync_copy(fb, m_sp.at[dstb])

        def eb(k, carry):
            base = (k * NS + s) * C
            pltpu.sync_copy(edge_dst.at[pl.ds(base, C)], dstb)
            pltpu.sync_copy(fb, deg_sp.at[dstb], add=True)
            return carry

        lax.fori_loop(0, EDGE_PER_TILE, eb, 0)

        @pl.when(s == NS - 1)
        def _():
            pltpu.sync_copy(edge_dst.at[pl.ds(EDGE_FULL * C, EDGE_REM)], dstr)
            pltpu.sync_copy(oner, deg_sp.at[dstr], add=True)

        zi16 = jnp.zeros((16,), _i32)
        dummy16 = jnp.full((16,), DUMMY, _i32)

        for rg in range(NRANGE):
            lo = rg * RANGE

            def compact_chunk(dref, sref, sz, state):
                off, flushed = state

                def vb(v, o):
                    d = dref[pl.ds(v * 16, 16)] - lo
                    m = (d >= 0) & (d < RANGE)
                    plsc.store_compressed(stg_s.at[pl.ds(o, 16)],
                                          sref[pl.ds(v * 16, 16)], mask=m)
                    plsc.store_compressed(stg_r.at[pl.ds(o, 16)], d, mask=m)
                    return o + jnp.sum(m.astype(_i32))

                off = lax.fori_loop(0, sz // 16, vb, off)

                def do_flush(st):
                    o, fl = st
                    dst = (rg * NS + s) * REGION + fl * C
                    pltpu.sync_copy(stg_s.at[pl.ds(0, C)],
                                    csrc.at[pl.ds(dst, C)])
                    pltpu.sync_copy(stg_r.at[pl.ds(0, C)],
                                    crel.at[pl.ds(dst, C)])

                    def mv(v, carry):
                        stg_s[pl.ds(v * 16, 16)] = stg_s[pl.ds(C + v * 16, 16)]
                        stg_r[pl.ds(v * 16, 16)] = stg_r[pl.ds(C + v * 16, 16)]
                        return carry

                    lax.fori_loop(0, C // 16, mv, 0)
                    return (o - C, fl + 1)

                return lax.cond(off >= C, do_flush, lambda st: st,
                                (off, flushed))

            def cb(k, state):
                base = (k * NS + s) * C
                pltpu.sync_copy(edge_dst.at[pl.ds(base, C)], dstb)
                pltpu.sync_copy(edge_src.at[pl.ds(base, C)], srcb)
                return compact_chunk(dstb, srcb, C, state)

            state = lax.fori_loop(0, EDGE_PER_TILE, cb, (0, 0))

            def tail(state):
                pltpu.sync_copy(edge_dst.at[pl.ds(EDGE_FULL * C, EDGE_REM)],
                                dstr)
                pltpu.sync_copy(edge_src.at[pl.ds(EDGE_FULL * C, EDGE_REM)],
                                srcr)
                return compact_chunk(dstr, srcr, EDGE_REM, state)

            state = lax.cond(s == NS - 1, tail, lambda st: st, state)
            off, flushed = state

            def pad(v, carry):
                stg_s[pl.ds(off + v * 16, 16)] = zi16
                stg_r[pl.ds(off + v * 16, 16)] = dummy16
                return carry

            lax.fori_loop(0, C // 16, pad, 0)
            dst = (rg * NS + s) * REGION + flushed * C
            pltpu.sync_copy(stg_s.at[pl.ds(0, C)], csrc.at[pl.ds(dst, C)])
            pltpu.sync_copy(stg_r.at[pl.ds(0, C)], crel.at[pl.ds(dst, C)])
            flushed = flushed + 1

            cntb[pl.ds(0, 16)] = jnp.full((16,), 1, _i32) * flushed
            pltpu.sync_copy(cntb.at[pl.ds(0, CNT_STRIDE)],
                            cnt.at[pl.ds((rg * NS + s) * CNT_STRIDE,
                                         CNT_STRIDE)])

        plsc.subcore_barrier()

        def node_chunk(base, sz):
            pltpu.sync_copy(m_sp.at[pl.ds(base, sz)], mb.at[pl.ds(0, sz)])
            pltpu.sync_copy(mb.at[pl.ds(0, sz)], m_out.at[pl.ds(base, sz)])
            pltpu.sync_copy(deg_sp.at[pl.ds(base, sz)], degb.at[pl.ds(0, sz)])

            def vb(v, carry):
                d = jnp.maximum(degb[pl.ds(v * 16, 16)], 1.0)
                ab[pl.ds(v * 16, 16)] = _rsqrt16(d)
                return carry

            lax.fori_loop(0, sz // 16, vb, 0)
            pltpu.sync_copy(ab.at[pl.ds(0, sz)], a_out.at[pl.ds(base, sz)])
            pltpu.sync_copy(emb.at[pl.ds(base, sz), :], rowb.at[pl.ds(0, sz), :])

            def rb(k, carry):
                av = ab[pl.ds(k * 16, 16)]
                for j in range(16):
                    r = k * 16 + j
                    sa = _lane_bcast(av, j)
                    rowb[r, pl.ds(0, 16)] = rowb[r, pl.ds(0, 16)] * sa
                    rowb[r, pl.ds(16, 16)] = rowb[r, pl.ds(16, 16)] * sa
                return carry

            lax.fori_loop(0, sz // 16, rb, 0)
            pltpu.sync_copy(rowb.at[pl.ds(0, sz), :],
                            h_out.at[pl.ds(base, sz), :])

        def nb(k, carry):
            g = k * NS + s

            @pl.when(g < NODE_FULL)
            def _():
                node_chunk(g * C, C)

            return carry

        lax.fori_loop(0, NODE_K, nb, 0)

        @pl.when(s == NS - 1)
        def _():
            node_chunk(NODE_FULL * C, NODE_REM)

    @pl.when(c == 0)
    def _():
        side(users, edge_u, edge_i, user_emb, a_u, h0_u, m_u,
             csrc_u, crel_u, cnt_u)

    @pl.when(c == 1)
    def _():
        side(items, edge_i, edge_u, item_emb, a_i, h0_i, m_i,
             csrc_i, crel_i, cnt_i)


@functools.lru_cache(maxsize=None)
def _k1():
    return pl.kernel(
        _k1_body,
        out_type=(
            jax.ShapeDtypeStruct((U,), _f32),
            jax.ShapeDtypeStruct((U,), _f32),
            jax.ShapeDtypeStruct((U, E), _f32),
            jax.ShapeDtypeStruct((U, E), _f32),
            jax.ShapeDtypeStruct((U,), _f32),
            jax.ShapeDtypeStruct((U,), _f32),
            jax.ShapeDtypeStruct((NRANGE * NS * REGION,), _i32),
            jax.ShapeDtypeStruct((NRANGE * NS * REGION,), _i32),
            jax.ShapeDtypeStruct((CNT_LEN,), _i32),
            jax.ShapeDtypeStruct((NRANGE * NS * REGION,), _i32),
            jax.ShapeDtypeStruct((NRANGE * NS * REGION,), _i32),
            jax.ShapeDtypeStruct((CNT_LEN,), _i32),
        ),
        mesh=_mesh(),
        compiler_params=_params,
        scratch_types=[
            pltpu.VMEM_SHARED((U,), _f32),
            pltpu.VMEM_SHARED((U,), _f32),
            pltpu.VMEM((C,), _i32),
            pltpu.VMEM((C,), _i32),
            pltpu.VMEM((C,), _f32),
            pltpu.VMEM((EDGE_REM,), _i32),
            pltpu.VMEM((EDGE_REM,), _i32),
            pltpu.VMEM((EDGE_REM,), _f32),
            pltpu.VMEM((C,), _f32),
            pltpu.VMEM((C,), _f32),
            pltpu.VMEM((C, E), _f32),
            pltpu.VMEM((C,), _f32),
            pltpu.VMEM((3 * C,), _i32),
            pltpu.VMEM((3 * C,), _i32),
            pltpu.VMEM((16,), _i32),
        ],
    )


def _layer_body(write_h, scale,
                csrc_u, crel_u, cnt_u, csrc_i, crel_i, cnt_i,
                a_u, a_i, h_u, h_i, p_u, p_i, *refs):
    if write_h:
        (acc_u, acc_i, ho_u, ho_i,
         accsp, srcb0, srcb1, relb0, relb1, rowsb0, rowsb1, ab, prevb, cntv,
         sem) = refs
    else:
        (acc_u, acc_i,
         accsp, srcb0, srcb1, relb0, relb1, rowsb0, rowsb1, ab, prevb, cntv,
         sem) = refs
        ho_u = ho_i = None
    srcbs = (srcb0, srcb1)
    relbs = (relb0, relb1)
    rowsbs = (rowsb0, rowsb1)

    c = lax.axis_index("c")
    s = lax.axis_index("s")

    def side(csrc, crel, cnt, srctab, a_tab, prev, acc_out, h_out):
        for rg in range(NRANGE):
            base_node = rg * RANGE

            _zero_rows(rowsb0, WB)
            for j in range(3):
                pltpu.sync_copy(rowsb0.at[pl.ds(0, WB), :],
                                accsp.at[pl.ds(s * 1564 + j * WB, WB), :])
            pltpu.sync_copy(rowsb0.at[pl.ds(0, 28), :],
                            accsp.at[pl.ds(s * 1564 + 3 * WB, 28), :])
            plsc.subcore_barrier()

            pltpu.sync_copy(cnt.at[pl.ds((rg * NS + s) * CNT_STRIDE, 16)],
                            cntv)
            lanes = lax.iota(_i32, 16)
            cvec = jnp.where(lanes < CNT_STRIDE, cntv[pl.ds(0, 16)], 0)
            n_chunks = jnp.max(cvec)

            def chunk(k, carry):
                base = (rg * NS + s) * REGION + k * C
                descs = []
                for b in range(2):
                    pltpu.sync_copy(
                        csrc.at[pl.ds(base + b * WB, WB)], srcbs[b])
                    descs.append(pltpu.async_copy(
                        srctab.at[srcbs[b]], rowsbs[b], sem))
                    pltpu.sync_copy(
                        crel.at[pl.ds(base + b * WB, WB)], relbs[b])
                for b in range(2):
                    descs[b].wait()
                for b in range(2):
                    pltpu.sync_copy(rowsbs[b], accsp.at[relbs[b]], add=True)
                return carry

            lax.fori_loop(0, n_chunks, chunk, 0)
            plsc.subcore_barrier()

            def wchunk(base_r, sz):
                pltpu.sync_copy(accsp.at[pl.ds(base_r, sz), :],
                                rowsb0.at[pl.ds(0, sz), :])
                pltpu.sync_copy(a_tab.at[pl.ds(base_node + base_r, sz)],
                                ab.at[pl.ds(0, sz)])
                pltpu.sync_copy(prev.at[pl.ds(base_node + base_r, sz), :],
                                prevb.at[pl.ds(0, sz), :])

                def rb_(k, carry):
                    av = ab[pl.ds(k * 16, 16)]
                    for j in range(16):
                        r = k * 16 + j
                        sa = _lane_bcast(av, j)
                        e0 = rowsb0[r, pl.ds(0, 16)] * sa
                        e1 = rowsb0[r, pl.ds(16, 16)] * sa
                        if write_h:
                            rowsb0[r, pl.ds(0, 16)] = e0 * sa
                            rowsb0[r, pl.ds(16, 16)] = e1 * sa
                        prevb[r, pl.ds(0, 16)] = (
                            prevb[r, pl.ds(0, 16)] + e0) * scale
                        prevb[r, pl.ds(16, 16)] = (
                            prevb[r, pl.ds(16, 16)] + e1) * scale
                    return carry

                lax.fori_loop(0, (sz + 15) // 16, rb_, 0)
                pltpu.sync_copy(
                    prevb.at[pl.ds(0, sz), :],
                    acc_out.at[pl.ds(base_node + base_r, sz), :])
                if write_h:
                    pltpu.sync_copy(
                        rowsb0.at[pl.ds(0, sz), :],
                        h_out.at[pl.ds(base_node + base_r, sz), :])

            def wb_(k, carry):
                wchunk((k * NS + s) * WB, WB)
                return carry

            lax.fori_loop(0, WB_K, wb_, 0)

            @pl.when(s == NS - 1)
            def _():
                wchunk(WB_FULL * WB, WB_REM)

            plsc.subcore_barrier()

    @pl.when(c == 0)
    def _():
        side(csrc_u, crel_u, cnt_u, h_i, a_u, p_u, acc_u, ho_u)

    @pl.when(c == 1)
    def _():
        side(csrc_i, crel_i, cnt_i, h_u, a_i, p_i, acc_i, ho_i)


def _layer_body_k2(*args):
    return _layer_body(True, 1.0, *args)


def _layer_body_k3(*args):
    return _layer_body(False, 1.0 / 3.0, *args)


@functools.lru_cache(maxsize=None)
def _make_layer(write_h):
    outs = [jax.ShapeDtypeStruct((U, E), _f32)] * 2
    if write_h:
        outs += [jax.ShapeDtypeStruct((U, E), _f32)] * 2
    return pl.kernel(
        _layer_body_k2 if write_h else _layer_body_k3,
        out_type=tuple(outs),
        mesh=_mesh(),
        compiler_params=_params,
        scratch_types=[
            pltpu.VMEM_SHARED((ACC_ROWS, E), _f32),
            pltpu.VMEM((WB,), _i32),
            pltpu.VMEM((WB,), _i32),
            pltpu.VMEM((WB,), _i32),
            pltpu.VMEM((WB,), _i32),
            pltpu.VMEM((WB, E), _f32),
            pltpu.VMEM((WB, E), _f32),
            pltpu.VMEM((WB,), _f32),
            pltpu.VMEM((WB, E), _f32),
            pltpu.VMEM((16,), _i32),
            pltpu.SemaphoreType.DMA,
        ],
    )


def _k4_body(users, items, f_u, f_i, u_his, i_his,
             ug_out, ig_out, ut_out, it_out,
             idxb, ongb, hisb, sem):
    c = lax.axis_index("c")
    s = lax.axis_index("s")

    def side(bidx, ftab, his, g_out, t_out):
        pltpu.sync_copy(bidx.at[pl.ds(s * C, C)], idxb)
        pltpu.async_copy(ftab.at[idxb], ongb, sem).wait()
        pltpu.async_copy(his.at[idxb], hisb, sem).wait()

        def rb(r, carry):
            o0 = ongb[r, pl.ds(0, 16)]
            o1 = ongb[r, pl.ds(16, 16)]
            hisb[r, pl.ds(0, 16)] = hisb[r, pl.ds(0, 16)] * MOM + o0 * (1.0 - MOM)
            hisb[r, pl.ds(16, 16)] = hisb[r, pl.ds(16, 16)] * MOM + o1 * (1.0 - MOM)
            return carry

        lax.fori_loop(0, C, rb, 0)
        pltpu.sync_copy(ongb, g_out.at[pl.ds(s * C, C), :])
        pltpu.sync_copy(hisb, t_out.at[pl.ds(s * C, C), :])

    @pl.when(c == 0)
    def _():
        side(users, f_u, u_his, ug_out, ut_out)

    @pl.when(c == 1)
    def _():
        side(items, f_i, i_his, ig_out, it_out)


@functools.lru_cache(maxsize=None)
def _k4():
    return pl.kernel(
        _k4_body,
        out_type=(
            jax.ShapeDtypeStruct((B, E), _f32),
            jax.ShapeDtypeStruct((B, E), _f32),
            jax.ShapeDtypeStruct((B, E), _f32),
            jax.ShapeDtypeStruct((B, E), _f32),
        ),
        mesh=_mesh(),
        compiler_params=_params,
        scratch_types=[
            pltpu.VMEM((C,), _i32),
            pltpu.VMEM((C, E), _f32),
            pltpu.VMEM((C, E), _f32),
            pltpu.SemaphoreType.DMA,
        ],
    )


def _pred_body(x_ref, w_ref, b_ref, o_ref):
    o_ref[...] = x_ref[...] @ w_ref[...].T + b_ref[...][None, :]


def _pred(x, W, b):
    blk = 2048
    return pl.pallas_call(
        _pred_body,
        grid=(B // blk,),
        in_specs=[
            pl.BlockSpec((blk, E), lambda i: (i, 0)),
            pl.BlockSpec((E, E), lambda i: (0, 0)),
            pl.BlockSpec((E,), lambda i: (0,)),
        ],
        out_specs=pl.BlockSpec((blk, E), lambda i: (i, 0)),
        out_shape=jax.ShapeDtypeStruct((B, E), _f32),
    )(x, W, b)


def _merge_body(f_ref, his_ref, m_ref, o_ref):
    o_ref[...] = jnp.where(m_ref[...] > 0.5, f_ref[...], his_ref[...])


def _merge(f, his, m):
    blk = 2000
    m32 = jnp.broadcast_to(m[:, None], (U, E))
    return pl.pallas_call(
        _merge_body,
        grid=(U // blk,),
        in_specs=[
            pl.BlockSpec((blk, E), lambda i: (i, 0)),
            pl.BlockSpec((blk, E), lambda i: (i, 0)),
            pl.BlockSpec((blk, E), lambda i: (i, 0)),
        ],
        out_specs=pl.BlockSpec((blk, E), lambda i: (i, 0)),
        out_shape=jax.ShapeDtypeStruct((U, E), _f32),
    )(f, his, m32)


def kernel(users, items, user_emb, item_emb, u_his, i_his, pred_W, pred_b,
           edge_u, edge_i):
    (a_u, a_i, h0_u, h0_i, m_u, m_i,
     csrc_u, crel_u, cnt_u, csrc_i, crel_i, cnt_i) = _k1()(
        users, items, user_emb, item_emb, edge_u, edge_i)
    acc_u, acc_i, h1_u, h1_i = _make_layer(True)(
        csrc_u, crel_u, cnt_u, csrc_i, crel_i, cnt_i, a_u, a_i,
        h0_u, h0_i, user_emb, item_emb)
    f_u, f_i = _make_layer(False)(
        csrc_u, crel_u, cnt_u, csrc_i, crel_i, cnt_i, a_u, a_i,
        h1_u, h1_i, acc_u, acc_i)
    u_on_g, i_on_g, u_target, i_target = _k4()(
        users, items, f_u, f_i, u_his, i_his)
    new_u_his = _merge(f_u, u_his, m_u)
    new_i_his = _merge(f_i, i_his, m_i)
    u_pred = _pred(u_on_g, pred_W, pred_b)
    i_pred = _pred(i_on_g, pred_W, pred_b)
    return (u_pred, u_target, i_pred, i_target, new_u_his, new_i_his)


# speedup vs baseline: 1.1775x; 1.1775x over previous
import functools

import jax
import jax.numpy as jnp
from jax import lax
from jax.experimental import pallas as pl
from jax.experimental.pallas import tpu as pltpu
from jax.experimental.pallas import tpu_sc as plsc

U = 100000
E = 32
NE = 1_000_000
B = 16384
MOM = 0.05
NS = 16
C = 1024

EDGE_FULL = NE // C
EDGE_PER_TILE = EDGE_FULL // NS
EDGE_REM = NE - EDGE_FULL * C
NODE_FULL = U // C
NODE_K = -(-NODE_FULL // NS)
NODE_REM = U - NODE_FULL * C

NRANGE = 4
RANGE = U // NRANGE
DUMMY = RANGE
ACC_ROWS = RANGE + 24
CAP_CHUNKS = 64
REGION = CAP_CHUNKS * C
WB = 512
WB_FULL = RANGE // WB
WB_K = WB_FULL // NS
WB_REM = RANGE - WB_FULL * WB

CNT_STRIDE = 8
CNT_LEN = NRANGE * NS * CNT_STRIDE + 8

_f32 = jnp.float32
_i32 = jnp.int32


@functools.lru_cache(maxsize=None)
def _mesh():
    return plsc.VectorSubcoreMesh(core_axis_name="c", subcore_axis_name="s",
                                  num_cores=2, num_subcores=NS)


_params = pltpu.CompilerParams(needs_layout_passes=False,
                               use_tc_tiling_on_sc=False)


def _rsqrt16(d):
    xi = plsc.bitcast(d, _i32)
    y = plsc.bitcast(jnp.int32(0x5F3759DF) - (xi >> 1), _f32)
    for _ in range(3):
        y = y * (1.5 - 0.5 * d * y * y)
    return y


def _lane_bcast(v, j):
    idx = jnp.full((16, 1), j, dtype=_i32)
    dnums = lax.GatherDimensionNumbers(
        offset_dims=(), collapsed_slice_dims=(0,), start_index_map=(0,))
    return lax.gather(v, idx, dnums, (1,),
                      mode=lax.GatherScatterMode.PROMISE_IN_BOUNDS)


def _fill(ref, n, value):
    val = jnp.full((16,), value, _f32)

    def body(v, carry):
        ref[pl.ds(v * 16, 16)] = val
        return carry

    lax.fori_loop(0, n // 16, body, 0)


def _zero_rows(ref, rows):
    z = jnp.zeros((16,), _f32)

    def body(r, carry):
        ref[r, pl.ds(0, 16)] = z
        ref[r, pl.ds(16, 16)] = z
        return carry

    lax.fori_loop(0, rows, body, 0)


def _k1_body(users, items, user_emb, item_emb, edge_u, edge_i,
             a_u, a_i, h0_u, h0_i, m_u, m_i,
             csrc_u, crel_u, cnt_u, csrc_i, crel_i, cnt_i,
             deg_sp, m_sp, dstb, srcb, fb, dstr, srcr, oner,
             degb, ab, rowb, mb, stg_s0, stg_s1, stg_s2, stg_s3,
             stg_r0, stg_r1, stg_r2, stg_r3, cntb):
    c = lax.axis_index("c")
    s = lax.axis_index("s")

    def side(bidx, edge_dst, edge_src, emb, a_out, h_out, m_out,
             csrc, crel, cnt):
        _fill(fb, C, 0.0)

        def zb(k, carry):
            g = k * NS + s

            @pl.when(g < NODE_FULL)
            def _():
                pltpu.sync_copy(fb, deg_sp.at[pl.ds(g * C, C)])
                pltpu.sync_copy(fb, m_sp.at[pl.ds(g * C, C)])

            return carry

        lax.fori_loop(0, NODE_K, zb, 0)

        @pl.when(s == NS - 1)
        def _():
            pltpu.sync_copy(fb.at[pl.ds(0, NODE_REM)],
                            deg_sp.at[pl.ds(NODE_FULL * C, NODE_REM)])
            pltpu.sync_copy(fb.at[pl.ds(0, NODE_REM)],
                            m_sp.at[pl.ds(NODE_FULL * C, NODE_REM)])

        plsc.subcore_barrier()

        _fill(fb, C, 1.0)
        _fill(oner, EDGE_REM, 1.0)

        pltpu.sync_copy(bidx.at[pl.ds(s * C, C)], dstb)
        pltpu.sync_copy(fb, m_sp.at[dstb])

        zi16 = jnp.zeros((16,), _i32)
        dummy16 = jnp.full((16,), DUMMY, _i32)
        stgs_s = (stg_s0, stg_s1, stg_s2, stg_s3)
        stgs_r = (stg_r0, stg_r1, stg_r2, stg_r3)

        def compact_chunk(dref, sref, sz, state):
            offs, fls = state[:NRANGE], state[NRANGE:]

            def vb(v, offs):
                d0 = dref[pl.ds(v * 16, 16)]
                sv = sref[pl.ds(v * 16, 16)]
                new = []
                for rg in range(NRANGE):
                    d = d0 - rg * RANGE
                    m = (d >= 0) & (d < RANGE)
                    plsc.store_compressed(
                        stgs_s[rg].at[pl.ds(offs[rg], 16)], sv, mask=m)
                    plsc.store_compressed(
                        stgs_r[rg].at[pl.ds(offs[rg], 16)], d, mask=m)
                    new.append(offs[rg] + jnp.sum(m.astype(_i32)))
                return tuple(new)

            offs = lax.fori_loop(0, sz // 16, vb, offs)

            out_o, out_f = [], []
            for rg in range(NRANGE):
                def do_flush(st, rg=rg):
                    o2, f2 = st
                    dst = (rg * NS + s) * REGION + f2 * C
                    pltpu.sync_copy(stgs_s[rg].at[pl.ds(0, C)],
                                    csrc.at[pl.ds(dst, C)])
                    pltpu.sync_copy(stgs_r[rg].at[pl.ds(0, C)],
                                    crel.at[pl.ds(dst, C)])

                    def mv(v, carry):
                        stgs_s[rg][pl.ds(v * 16, 16)] = (
                            stgs_s[rg][pl.ds(C + v * 16, 16)])
                        stgs_r[rg][pl.ds(v * 16, 16)] = (
                            stgs_r[rg][pl.ds(C + v * 16, 16)])
                        return carry

                    lax.fori_loop(0, C // 16, mv, 0)
                    return (o2 - C, f2 + 1)

                o, fl = lax.cond(offs[rg] >= C, do_flush, lambda st: st,
                                 (offs[rg], fls[rg]))
                out_o.append(o)
                out_f.append(fl)
            return tuple(out_o) + tuple(out_f)

        def cb(k, state):
            base = (k * NS + s) * C
            pltpu.sync_copy(edge_dst.at[pl.ds(base, C)], dstb)
            pltpu.sync_copy(edge_src.at[pl.ds(base, C)], srcb)
            pltpu.sync_copy(fb, deg_sp.at[dstb], add=True)
            return compact_chunk(dstb, srcb, C, state)

        state = lax.fori_loop(0, EDGE_PER_TILE, cb, (0,) * (2 * NRANGE))

        def tail(state):
            pltpu.sync_copy(edge_dst.at[pl.ds(EDGE_FULL * C, EDGE_REM)], dstr)
            pltpu.sync_copy(edge_src.at[pl.ds(EDGE_FULL * C, EDGE_REM)], srcr)
            pltpu.sync_copy(oner, deg_sp.at[dstr], add=True)
            return compact_chunk(dstr, srcr, EDGE_REM, state)

        state = lax.cond(s == NS - 1, tail, lambda st: st, state)

        for rg in range(NRANGE):
            off, flushed = state[rg], state[NRANGE + rg]

            def pad(v, carry):
                stgs_s[rg][pl.ds(off + v * 16, 16)] = zi16
                stgs_r[rg][pl.ds(off + v * 16, 16)] = dummy16
                return carry

            lax.fori_loop(0, C // 16, pad, 0)
            dst = (rg * NS + s) * REGION + flushed * C
            pltpu.sync_copy(stgs_s[rg].at[pl.ds(0, C)], csrc.at[pl.ds(dst, C)])
            pltpu.sync_copy(stgs_r[rg].at[pl.ds(0, C)], crel.at[pl.ds(dst, C)])
            flushed = flushed + 1

            cntb[pl.ds(0, 16)] = jnp.full((16,), 1, _i32) * flushed
            pltpu.sync_copy(cntb.at[pl.ds(0, CNT_STRIDE)],
                            cnt.at[pl.ds((rg * NS + s) * CNT_STRIDE,
                                         CNT_STRIDE)])

        plsc.subcore_barrier()

        def node_chunk(base, sz):
            pltpu.sync_copy(m_sp.at[pl.ds(base, sz)], mb.at[pl.ds(0, sz)])
            pltpu.sync_copy(mb.at[pl.ds(0, sz)], m_out.at[pl.ds(base, sz)])
            pltpu.sync_copy(deg_sp.at[pl.ds(base, sz)], degb.at[pl.ds(0, sz)])

            def vb(v, carry):
                d = jnp.maximum(degb[pl.ds(v * 16, 16)], 1.0)
                ab[pl.ds(v * 16, 16)] = _rsqrt16(d)
                return carry

            lax.fori_loop(0, sz // 16, vb, 0)
            pltpu.sync_copy(ab.at[pl.ds(0, sz)], a_out.at[pl.ds(base, sz)])
            pltpu.sync_copy(emb.at[pl.ds(base, sz), :], rowb.at[pl.ds(0, sz), :])

            def rb(k, carry):
                av = ab[pl.ds(k * 16, 16)]
                for j in range(16):
                    r = k * 16 + j
                    sa = _lane_bcast(av, j)
                    rowb[r, pl.ds(0, 16)] = rowb[r, pl.ds(0, 16)] * sa
                    rowb[r, pl.ds(16, 16)] = rowb[r, pl.ds(16, 16)] * sa
                return carry

            lax.fori_loop(0, sz // 16, rb, 0)
            pltpu.sync_copy(rowb.at[pl.ds(0, sz), :],
                            h_out.at[pl.ds(base, sz), :])

        def nb(k, carry):
            g = k * NS + s

            @pl.when(g < NODE_FULL)
            def _():
                node_chunk(g * C, C)

            return carry

        lax.fori_loop(0, NODE_K, nb, 0)

        @pl.when(s == NS - 1)
        def _():
            node_chunk(NODE_FULL * C, NODE_REM)

    @pl.when(c == 0)
    def _():
        side(users, edge_u, edge_i, user_emb, a_u, h0_u, m_u,
             csrc_u, crel_u, cnt_u)

    @pl.when(c == 1)
    def _():
        side(items, edge_i, edge_u, item_emb, a_i, h0_i, m_i,
             csrc_i, crel_i, cnt_i)


@functools.lru_cache(maxsize=None)
def _k1():
    return pl.kernel(
        _k1_body,
        out_type=(
            jax.ShapeDtypeStruct((U,), _f32),
            jax.ShapeDtypeStruct((U,), _f32),
            jax.ShapeDtypeStruct((U, E), _f32),
            jax.ShapeDtypeStruct((U, E), _f32),
            jax.ShapeDtypeStruct((U,), _f32),
            jax.ShapeDtypeStruct((U,), _f32),
            jax.ShapeDtypeStruct((NRANGE * NS * REGION,), _i32),
            jax.ShapeDtypeStruct((NRANGE * NS * REGION,), _i32),
            jax.ShapeDtypeStruct((CNT_LEN,), _i32),
            jax.ShapeDtypeStruct((NRANGE * NS * REGION,), _i32),
            jax.ShapeDtypeStruct((NRANGE * NS * REGION,), _i32),
            jax.ShapeDtypeStruct((CNT_LEN,), _i32),
        ),
        mesh=_mesh(),
        compiler_params=_params,
        scratch_types=[
            pltpu.VMEM_SHARED((U,), _f32),
            pltpu.VMEM_SHARED((U,), _f32),
            pltpu.VMEM((C,), _i32),
            pltpu.VMEM((C,), _i32),
            pltpu.VMEM((C,), _f32),
            pltpu.VMEM((EDGE_REM,), _i32),
            pltpu.VMEM((EDGE_REM,), _i32),
            pltpu.VMEM((EDGE_REM,), _f32),
            pltpu.VMEM((C,), _f32),
            pltpu.VMEM((C,), _f32),
            pltpu.VMEM((C, E), _f32),
            pltpu.VMEM((C,), _f32),
            pltpu.VMEM((3 * C,), _i32),
            pltpu.VMEM((3 * C,), _i32),
            pltpu.VMEM((3 * C,), _i32),
            pltpu.VMEM((3 * C,), _i32),
            pltpu.VMEM((3 * C,), _i32),
            pltpu.VMEM((3 * C,), _i32),
            pltpu.VMEM((3 * C,), _i32),
            pltpu.VMEM((3 * C,), _i32),
            pltpu.VMEM((16,), _i32),
        ],
    )


def _layer_body(write_h, scale,
                csrc_u, crel_u, cnt_u, csrc_i, crel_i, cnt_i,
                a_u, a_i, h_u, h_i, p_u, p_i, *refs):
    if write_h:
        (acc_u, acc_i, ho_u, ho_i,
         accsp, srcb, relb, rowsb, ab, prevb, cntv, sem) = refs
    else:
        (acc_u, acc_i,
         accsp, srcb, relb, rowsb, ab, prevb, cntv, sem) = refs
        ho_u = ho_i = None

    c = lax.axis_index("c")
    s = lax.axis_index("s")

    def side(csrc, crel, cnt, srctab, a_tab, prev, acc_out, h_out):
        for rg in range(NRANGE):
            base_node = rg * RANGE

            _zero_rows(rowsb, WB)
            for j in range(3):
                pltpu.sync_copy(rowsb.at[pl.ds(0, WB), :],
                                accsp.at[pl.ds(s * 1564 + j * WB, WB), :])
            pltpu.sync_copy(rowsb.at[pl.ds(0, 28), :],
                            accsp.at[pl.ds(s * 1564 + 3 * WB, 28), :])
            plsc.subcore_barrier()

            pltpu.sync_copy(cnt.at[pl.ds((rg * NS + s) * CNT_STRIDE, 16)],
                            cntv)
            lanes = lax.iota(_i32, 16)
            cvec = jnp.where(lanes < CNT_STRIDE, cntv[pl.ds(0, 16)], 0)
            n_chunks = jnp.max(cvec)

            def chunk(k, carry):
                base = (rg * NS + s) * REGION + k * C
                pltpu.sync_copy(csrc.at[pl.ds(base, C)], srcb)
                d0 = pltpu.async_copy(srctab.at[srcb], rowsb, sem)
                pltpu.sync_copy(crel.at[pl.ds(base, C)], relb)
                d0.wait()
                pltpu.sync_copy(rowsb, accsp.at[relb], add=True)
                return carry

            lax.fori_loop(0, n_chunks, chunk, 0)
            plsc.subcore_barrier()

            def wchunk(base_r, sz):
                pltpu.sync_copy(accsp.at[pl.ds(base_r, sz), :],
                                rowsb.at[pl.ds(0, sz), :])
                pltpu.sync_copy(a_tab.at[pl.ds(base_node + base_r, sz)],
                                ab.at[pl.ds(0, sz)])
                pltpu.sync_copy(prev.at[pl.ds(base_node + base_r, sz), :],
                                prevb.at[pl.ds(0, sz), :])

                def rb_(k, carry):
                    av = ab[pl.ds(k * 16, 16)]
                    for j in range(16):
                        r = k * 16 + j
                        sa = _lane_bcast(av, j)
                        e0 = rowsb[r, pl.ds(0, 16)] * sa
                        e1 = rowsb[r, pl.ds(16, 16)] * sa
                        if write_h:
                            rowsb[r, pl.ds(0, 16)] = e0 * sa
                            rowsb[r, pl.ds(16, 16)] = e1 * sa
                        prevb[r, pl.ds(0, 16)] = (
                            prevb[r, pl.ds(0, 16)] + e0) * scale
                        prevb[r, pl.ds(16, 16)] = (
                            prevb[r, pl.ds(16, 16)] + e1) * scale
                    return carry

                lax.fori_loop(0, (sz + 15) // 16, rb_, 0)
                pltpu.sync_copy(
                    prevb.at[pl.ds(0, sz), :],
                    acc_out.at[pl.ds(base_node + base_r, sz), :])
                if write_h:
                    pltpu.sync_copy(
                        rowsb.at[pl.ds(0, sz), :],
                        h_out.at[pl.ds(base_node + base_r, sz), :])

            def wb_(k, carry):
                wchunk((k * NS + s) * WB, WB)
                return carry

            lax.fori_loop(0, WB_K, wb_, 0)

            @pl.when(s == NS - 1)
            def _():
                wchunk(WB_FULL * WB, WB_REM)

            plsc.subcore_barrier()

    @pl.when(c == 0)
    def _():
        side(csrc_u, crel_u, cnt_u, h_i, a_u, p_u, acc_u, ho_u)

    @pl.when(c == 1)
    def _():
        side(csrc_i, crel_i, cnt_i, h_u, a_i, p_i, acc_i, ho_i)


def _layer_body_k2(*args):
    return _layer_body(True, 1.0, *args)


def _layer_body_k3(*args):
    return _layer_body(False, 1.0 / 3.0, *args)


@functools.lru_cache(maxsize=None)
def _make_layer(write_h):
    outs = [jax.ShapeDtypeStruct((U, E), _f32)] * 2
    if write_h:
        outs += [jax.ShapeDtypeStruct((U, E), _f32)] * 2
    return pl.kernel(
        _layer_body_k2 if write_h else _layer_body_k3,
        out_type=tuple(outs),
        mesh=_mesh(),
        compiler_params=_params,
        scratch_types=[
            pltpu.VMEM_SHARED((ACC_ROWS, E), _f32),
            pltpu.VMEM((C,), _i32),
            pltpu.VMEM((C,), _i32),
            pltpu.VMEM((C, E), _f32),
            pltpu.VMEM((WB,), _f32),
            pltpu.VMEM((WB, E), _f32),
            pltpu.VMEM((16,), _i32),
            pltpu.SemaphoreType.DMA,
        ],
    )


def _k4_body(users, items, f_u, f_i, u_his, i_his,
             ug_out, ig_out, ut_out, it_out,
             idxb, ongb, hisb, sem):
    c = lax.axis_index("c")
    s = lax.axis_index("s")

    def side(bidx, ftab, his, g_out, t_out):
        pltpu.sync_copy(bidx.at[pl.ds(s * C, C)], idxb)
        pltpu.async_copy(ftab.at[idxb], ongb, sem).wait()
        pltpu.async_copy(his.at[idxb], hisb, sem).wait()

        def rb(r, carry):
            o0 = ongb[r, pl.ds(0, 16)]
            o1 = ongb[r, pl.ds(16, 16)]
            hisb[r, pl.ds(0, 16)] = hisb[r, pl.ds(0, 16)] * MOM + o0 * (1.0 - MOM)
            hisb[r, pl.ds(16, 16)] = hisb[r, pl.ds(16, 16)] * MOM + o1 * (1.0 - MOM)
            return carry

        lax.fori_loop(0, C, rb, 0)
        pltpu.sync_copy(ongb, g_out.at[pl.ds(s * C, C), :])
        pltpu.sync_copy(hisb, t_out.at[pl.ds(s * C, C), :])

    @pl.when(c == 0)
    def _():
        side(users, f_u, u_his, ug_out, ut_out)

    @pl.when(c == 1)
    def _():
        side(items, f_i, i_his, ig_out, it_out)


@functools.lru_cache(maxsize=None)
def _k4():
    return pl.kernel(
        _k4_body,
        out_type=(
            jax.ShapeDtypeStruct((B, E), _f32),
            jax.ShapeDtypeStruct((B, E), _f32),
            jax.ShapeDtypeStruct((B, E), _f32),
            jax.ShapeDtypeStruct((B, E), _f32),
        ),
        mesh=_mesh(),
        compiler_params=_params,
        scratch_types=[
            pltpu.VMEM((C,), _i32),
            pltpu.VMEM((C, E), _f32),
            pltpu.VMEM((C, E), _f32),
            pltpu.SemaphoreType.DMA,
        ],
    )


def _pred_body(x_ref, w_ref, b_ref, o_ref):
    o_ref[...] = x_ref[...] @ w_ref[...].T + b_ref[...][None, :]


def _pred(x, W, b):
    blk = 2048
    return pl.pallas_call(
        _pred_body,
        grid=(B // blk,),
        in_specs=[
            pl.BlockSpec((blk, E), lambda i: (i, 0)),
            pl.BlockSpec((E, E), lambda i: (0, 0)),
            pl.BlockSpec((E,), lambda i: (0,)),
        ],
        out_specs=pl.BlockSpec((blk, E), lambda i: (i, 0)),
        out_shape=jax.ShapeDtypeStruct((B, E), _f32),
    )(x, W, b)


def _merge_body(f_ref, his_ref, m_ref, o_ref):
    o_ref[...] = jnp.where(m_ref[...] > 0.5, f_ref[...], his_ref[...])


def _merge(f, his, m):
    blk = 2000
    m32 = jnp.broadcast_to(m[:, None], (U, E))
    return pl.pallas_call(
        _merge_body,
        grid=(U // blk,),
        in_specs=[
            pl.BlockSpec((blk, E), lambda i: (i, 0)),
            pl.BlockSpec((blk, E), lambda i: (i, 0)),
            pl.BlockSpec((blk, E), lambda i: (i, 0)),
        ],
        out_specs=pl.BlockSpec((blk, E), lambda i: (i, 0)),
        out_shape=jax.ShapeDtypeStruct((U, E), _f32),
    )(f, his, m32)


def kernel(users, items, user_emb, item_emb, u_his, i_his, pred_W, pred_b,
           edge_u, edge_i):
    (a_u, a_i, h0_u, h0_i, m_u, m_i,
     csrc_u, crel_u, cnt_u, csrc_i, crel_i, cnt_i) = _k1()(
        users, items, user_emb, item_emb, edge_u, edge_i)
    acc_u, acc_i, h1_u, h1_i = _make_layer(True)(
        csrc_u, crel_u, cnt_u, csrc_i, crel_i, cnt_i, a_u, a_i,
        h0_u, h0_i, user_emb, item_emb)
    f_u, f_i = _make_layer(False)(
        csrc_u, crel_u, cnt_u, csrc_i, crel_i, cnt_i, a_u, a_i,
        h1_u, h1_i, acc_u, acc_i)
    u_on_g, i_on_g, u_target, i_target = _k4()(
        users, items, f_u, f_i, u_his, i_his)
    new_u_his = _merge(f_u, u_his, m_u)
    new_i_his = _merge(f_i, i_his, m_i)
    u_pred = _pred(u_on_g, pred_W, pred_b)
    i_pred = _pred(i_on_g, pred_W, pred_b)
    return (u_pred, u_target, i_pred, i_target, new_u_his, new_i_his)
